# baseline (speedup 1.0000x reference)
import functools

import jax
import jax.numpy as jnp
from jax import lax
from jax.experimental import pallas as pl
from jax.experimental.pallas import tpu as pltpu
from jax.experimental.pallas import tpu_sc as plsc

N = 10000
E = 320000
F = 128
O = 64

NUM_RANGES = 16
NUM_EHALF = 2
RNG = 626
NPAD = NUM_RANGES * RNG
EHALF = E // NUM_EHALF
CHUNK = 4000
NGROUPS = CHUNK // 16
NCHUNK = EHALF // CHUNK


def _pre_body(x_ref, wp_ref, bp_ref, h_ref, hp_ref):
    h = jnp.log(x_ref[...] + 1.0)
    h_ref[...] = h
    hp_ref[...] = jnp.maximum(h @ wp_ref[...] + bp_ref[...], 0.0)


def _pre(x, W_pool, b_pool):
    blk = 1000
    grid = (N // blk,)
    return pl.pallas_call(
        _pre_body,
        grid=grid,
        in_specs=[
            pl.BlockSpec((blk, F), lambda i: (i, 0)),
            pl.BlockSpec((F, F), lambda i: (0, 0)),
            pl.BlockSpec((1, F), lambda i: (0, 0)),
        ],
        out_specs=[
            pl.BlockSpec((blk, F), lambda i: (i, 0)),
            pl.BlockSpec((blk, F), lambda i: (i, 0)),
        ],
        out_shape=[
            jax.ShapeDtypeStruct((N, F), jnp.float32),
            jax.ShapeDtypeStruct((N, F), jnp.float32),
        ],
    )(x, W_pool, b_pool.reshape(1, F))


def _segmax_body(hp_hbm, src_hbm, dst_hbm, out_hbm,
                 accum, sbuf0, dbuf0, sbuf1, dbuf1, csrc, cdst,
                 rows0, rows1, rows2, rows3,
                 sem_s0, sem_d0, sem_s1, sem_d1,
                 sem_g0, sem_g1, sem_g2, sem_g3):
    nc = lax.axis_index("c")
    ns = lax.axis_index("s")
    wid = ns * 2 + nc
    rid = wid % NUM_RANGES
    eh = wid // NUM_RANGES
    lo = rid * RNG
    trash = RNG

    zero16 = jnp.zeros((16,), jnp.float32)

    def _z(i, _):
        accum[pl.ds(i * 16, 16)] = zero16
        return 0

    lax.fori_loop(0, (RNG + 1) * F // 16, _z, 0, unroll=8)

    ebase = eh * EHALF

    def stage(c, sb, db, ss, sd):
        off = ebase + c * CHUNK
        pltpu.async_copy(src_hbm.at[pl.ds(off, CHUNK)], sb, ss)
        pltpu.async_copy(dst_hbm.at[pl.ds(off, CHUNK)], db, sd)

    def work(sbuf, dbuf, sem_s, sem_d):
        pltpu.make_async_copy(src_hbm.at[pl.ds(0, CHUNK)], sbuf, sem_s).wait()
        pltpu.make_async_copy(dst_hbm.at[pl.ds(0, CHUNK)], dbuf, sem_d).wait()

        def scan_body(g, n):
            sv = sbuf[pl.ds(g * 16, 16)]
            dv = dbuf[pl.ds(g * 16, 16)]
            d0 = dv - lo
            mi = ((d0 >> 31) + 1) & (((RNG - 1 - d0) >> 31) + 1)
            pos = plsc.cumsum(mi)
            tgt = (CHUNK + 16) + mi * (n + pos - 1 - (CHUNK + 16))
            plsc.store_scatter(csrc, [tgt], sv)
            plsc.store_scatter(cdst, [tgt], d0)
            return n + pos[15]

        n = lax.fori_loop(0, NGROUPS, scan_body, jnp.int32(0), unroll=4)

        csrc[pl.ds(n, 16)] = jnp.zeros((16,), jnp.int32)
        cdst[pl.ds(n, 16)] = jnp.full((16,), trash, jnp.int32)
        ngroups = (n + 15) // 16

        def issue(g, rows, sem):
            idxv = csrc[pl.ds(g * 16, 16)]
            pltpu.async_copy(hp_hbm.at[idxv], rows, sem)

        def rmw(g, rows, sem):
            pltpu.make_async_copy(hp_hbm.at[csrc[pl.ds(g * 16, 16)]],
                                  rows, sem).wait()
            dvec = cdst[pl.ds(g * 16, 16)]
            for j in range(16):
                d = dvec[j]
                for f in range(F // 16):
                    a = accum[pl.ds(d * F + f * 16, 16)]
                    m = rows[j, pl.ds(f * 16, 16)]
                    accum[pl.ds(d * F + f * 16, 16)] = jnp.maximum(a, m)

        ring = ((rows0, sem_g0), (rows1, sem_g1),
                (rows2, sem_g2), (rows3, sem_g3))

        @pl.when(ngroups > 0)
        def _():
            for k in range(3):
                @pl.when(k < ngroups)
                def _(k=k):
                    issue(k, *ring[k])

            def quad_body(q, _):
                for j in range(4):
                    g = q * 4 + j

                    @pl.when(g + 3 < ngroups)
                    def _(g=g, j=j):
                        issue(g + 3, *ring[(j + 3) % 4])

                    @pl.when(g < ngroups)
                    def _(g=g, j=j):
                        rmw(g, *ring[j])
                return 0

            lax.fori_loop(0, (ngroups + 3) // 4, quad_body, 0)

    stage(0, sbuf0, dbuf0, sem_s0, sem_d0)

    def chunk_pair(p, _):
        c0 = p * 2
        stage(c0 + 1, sbuf1, dbuf1, sem_s1, sem_d1)
        work(sbuf0, dbuf0, sem_s0, sem_d0)

        @pl.when(c0 + 2 < NCHUNK)
        def _():
            stage(c0 + 2, sbuf0, dbuf0, sem_s0, sem_d0)
        work(sbuf1, dbuf1, sem_s1, sem_d1)
        return 0

    lax.fori_loop(0, NCHUNK // 2, chunk_pair, 0)

    pltpu.sync_copy(accum.at[pl.ds(0, RNG * F)],
                    out_hbm.at[eh, pl.ds(lo * F, RNG * F)])


def _segmax(hp, src, dst):
    mesh = plsc.VectorSubcoreMesh(core_axis_name="c", subcore_axis_name="s")
    kfn = pl.kernel(
        _segmax_body,
        out_type=jax.ShapeDtypeStruct((NUM_EHALF, NPAD * F), jnp.float32),
        mesh=mesh,
        compiler_params=pltpu.CompilerParams(needs_layout_passes=False),
        scratch_types=[
            pltpu.VMEM(((RNG + 1) * F,), jnp.float32),
            pltpu.VMEM((CHUNK,), jnp.int32),
            pltpu.VMEM((CHUNK,), jnp.int32),
            pltpu.VMEM((CHUNK,), jnp.int32),
            pltpu.VMEM((CHUNK,), jnp.int32),
            pltpu.VMEM((CHUNK + 32,), jnp.int32),
            pltpu.VMEM((CHUNK + 32,), jnp.int32),
            pltpu.VMEM((16, F), jnp.float32),
            pltpu.VMEM((16, F), jnp.float32),
            pltpu.VMEM((16, F), jnp.float32),
            pltpu.VMEM((16, F), jnp.float32),
            pltpu.SemaphoreType.DMA,
            pltpu.SemaphoreType.DMA,
            pltpu.SemaphoreType.DMA,
            pltpu.SemaphoreType.DMA,
            pltpu.SemaphoreType.DMA,
            pltpu.SemaphoreType.DMA,
            pltpu.SemaphoreType.DMA,
            pltpu.SemaphoreType.DMA,
        ],
    )
    return kfn(hp, src, dst)


def _post_body(h_ref, p0_ref, p1_ref, ws_ref, wn_ref, b_ref, o_ref):
    neigh = jnp.maximum(p0_ref[...], p1_ref[...])
    o_ref[...] = h_ref[...] @ ws_ref[...] + neigh @ wn_ref[...] + b_ref[...]


def _post(h, partial, W_self, W_neigh, bias):
    blk = 1000
    grid = (N // blk,)
    p = partial.reshape(NUM_EHALF, NPAD, F)
    return pl.pallas_call(
        _post_body,
        grid=grid,
        in_specs=[
            pl.BlockSpec((blk, F), lambda i: (i, 0)),
            pl.BlockSpec((blk, F), lambda i: (i, 0)),
            pl.BlockSpec((blk, F), lambda i: (i, 0)),
            pl.BlockSpec((F, O), lambda i: (0, 0)),
            pl.BlockSpec((F, O), lambda i: (0, 0)),
            pl.BlockSpec((1, O), lambda i: (0, 0)),
        ],
        out_specs=pl.BlockSpec((blk, O), lambda i: (i, 0)),
        out_shape=jax.ShapeDtypeStruct((N, O), jnp.float32),
    )(h, p[0, :N], p[1, :N], W_self, W_neigh, bias.reshape(1, O))


@jax.jit
def kernel(x, edge_index, W_pool, b_pool, W_self, W_neigh, bias):
    src = edge_index[0].astype(jnp.int32)
    dst = edge_index[1].astype(jnp.int32)
    h, hp = _pre(x, W_pool, b_pool)
    partial = _segmax(hp, src, dst)
    return _post(h, partial, W_self, W_neigh, bias)


# speedup vs baseline: 1.8480x; 1.8480x over previous
import jax
import jax.numpy as jnp
from jax import lax
from jax.experimental import pallas as pl
from jax.experimental.pallas import tpu as pltpu
from jax.experimental.pallas import tpu_sc as plsc

N = 10000
E = 320000
F = 128
FP = F // 2
O = 64

NUM_RANGES = 8
NUM_EPART = 4
RNG = 1252
NPAD = NUM_RANGES * RNG
EPART = E // NUM_EPART
CHUNK = 4000
NGROUPS = CHUNK // 16
NCHUNK = EPART // CHUNK


def _rne_hi(f):
    u = lax.bitcast_convert_type(f, jnp.uint32)
    r = u + jnp.uint32(0x7FFF) + ((u >> 16) & jnp.uint32(1))
    return r & jnp.uint32(0xFFFF0000)


def _pre_body(x_ref, wpe_ref, wpo_ref, bpe_ref, bpo_ref, h_ref, hpp_ref):
    h = jnp.log(x_ref[...] + 1.0)
    h_ref[...] = h
    hp_e = jnp.maximum(h @ wpe_ref[...] + bpe_ref[...], 0.0)
    hp_o = jnp.maximum(h @ wpo_ref[...] + bpo_ref[...], 0.0)
    word = (_rne_hi(hp_e) >> 16) | _rne_hi(hp_o)
    hpp_ref[...] = lax.bitcast_convert_type(word, jnp.float32)


def _pre(x, W_pool, b_pool):
    blk = 1000
    grid = (N // blk,)
    return pl.pallas_call(
        _pre_body,
        grid=grid,
        in_specs=[
            pl.BlockSpec((blk, F), lambda i: (i, 0)),
            pl.BlockSpec((F, FP), lambda i: (0, 0)),
            pl.BlockSpec((F, FP), lambda i: (0, 0)),
            pl.BlockSpec((1, FP), lambda i: (0, 0)),
            pl.BlockSpec((1, FP), lambda i: (0, 0)),
        ],
        out_specs=[
            pl.BlockSpec((blk, F), lambda i: (i, 0)),
            pl.BlockSpec((blk, FP), lambda i: (i, 0)),
        ],
        out_shape=[
            jax.ShapeDtypeStruct((N, F), jnp.float32),
            jax.ShapeDtypeStruct((N, FP), jnp.float32),
        ],
    )(x, W_pool[:, 0::2], W_pool[:, 1::2],
      b_pool[0::2].reshape(1, FP), b_pool[1::2].reshape(1, FP))


def _segmax_body(hp_hbm, src_hbm, dst_hbm, out_hbm,
                 accum, sbuf0, dbuf0, sbuf1, dbuf1, csrc, cdst, rows0, rows1,
                 sem_s0, sem_d0, sem_s1, sem_d1, sem_g0, sem_g1):
    nc = lax.axis_index("c")
    ns = lax.axis_index("s")
    wid = ns * 2 + nc
    rid = wid % NUM_RANGES
    eq = wid // NUM_RANGES
    lo = rid * RNG
    trash = RNG

    zero16 = jnp.zeros((16,), jnp.float32)

    def _z(i, _):
        accum[pl.ds(i * 16, 16)] = zero16
        return 0

    lax.fori_loop(0, (RNG + 1) * FP // 16, _z, 0, unroll=8)

    ebase = eq * EPART

    def stage(c, sb, db, ss, sd):
        off = ebase + c * CHUNK
        pltpu.async_copy(src_hbm.at[pl.ds(off, CHUNK)], sb, ss)
        pltpu.async_copy(dst_hbm.at[pl.ds(off, CHUNK)], db, sd)

    def work(sbuf, dbuf, sem_s, sem_d):
        pltpu.make_async_copy(src_hbm.at[pl.ds(0, CHUNK)], sbuf, sem_s).wait()
        pltpu.make_async_copy(dst_hbm.at[pl.ds(0, CHUNK)], dbuf, sem_d).wait()

        def scan_body(g, n):
            sv = sbuf[pl.ds(g * 16, 16)]
            dv = dbuf[pl.ds(g * 16, 16)]
            d0 = dv - lo
            mi = ((d0 >> 31) + 1) & (((RNG - 1 - d0) >> 31) + 1)
            pos = plsc.cumsum(mi)
            tgt = (CHUNK + 16) + mi * (n + pos - 1 - (CHUNK + 16))
            plsc.store_scatter(csrc, [tgt], sv)
            plsc.store_scatter(cdst, [tgt], d0)
            return n + pos[15]

        n = lax.fori_loop(0, NGROUPS, scan_body, jnp.int32(0), unroll=4)

        csrc[pl.ds(n, 16)] = jnp.zeros((16,), jnp.int32)
        cdst[pl.ds(n, 16)] = jnp.full((16,), trash, jnp.int32)
        ngroups = (n + 15) // 16

        def issue(g, rows, sem):
            idxv = csrc[pl.ds(g * 16, 16)]
            pltpu.async_copy(hp_hbm.at[idxv], rows, sem)

        def rmw(g, rows, sem):
            pltpu.make_async_copy(hp_hbm.at[csrc[pl.ds(g * 16, 16)]],
                                  rows, sem).wait()
            dvec = cdst[pl.ds(g * 16, 16)]
            for j in range(16):
                d = dvec[j]
                for f in range(FP // 16):
                    a = plsc.bitcast(accum[pl.ds(d * FP + f * 16, 16)],
                                     jnp.bfloat16)
                    m = plsc.bitcast(rows[j, pl.ds(f * 16, 16)], jnp.bfloat16)
                    accum[pl.ds(d * FP + f * 16, 16)] = plsc.bitcast(
                        jnp.maximum(a, m), jnp.float32)

        @pl.when(ngroups > 0)
        def _():
            issue(0, rows0, sem_g0)

            def pair_body(p, _):
                g0 = p * 2
                g1 = p * 2 + 1

                @pl.when(g1 < ngroups)
                def _():
                    issue(g1, rows1, sem_g1)
                rmw(g0, rows0, sem_g0)

                @pl.when(g1 < ngroups)
                def _():
                    @pl.when(g1 + 1 < ngroups)
                    def _():
                        issue(g1 + 1, rows0, sem_g0)
                    rmw(g1, rows1, sem_g1)
                return 0

            lax.fori_loop(0, (ngroups + 1) // 2, pair_body, 0)

    stage(0, sbuf0, dbuf0, sem_s0, sem_d0)

    def chunk_pair(p, _):
        c0 = p * 2
        stage(c0 + 1, sbuf1, dbuf1, sem_s1, sem_d1)
        work(sbuf0, dbuf0, sem_s0, sem_d0)

        @pl.when(c0 + 2 < NCHUNK)
        def _():
            stage(c0 + 2, sbuf0, dbuf0, sem_s0, sem_d0)
        work(sbuf1, dbuf1, sem_s1, sem_d1)
        return 0

    lax.fori_loop(0, NCHUNK // 2, chunk_pair, 0)

    pltpu.sync_copy(accum.at[pl.ds(0, RNG * FP)],
                    out_hbm.at[eq, pl.ds(lo * FP, RNG * FP)])


def _segmax(hp, src, dst):
    mesh = plsc.VectorSubcoreMesh(core_axis_name="c", subcore_axis_name="s")
    kfn = pl.kernel(
        _segmax_body,
        out_type=jax.ShapeDtypeStruct((NUM_EPART, NPAD * FP), jnp.float32),
        mesh=mesh,
        compiler_params=pltpu.CompilerParams(
            needs_layout_passes=False, use_tc_tiling_on_sc=False),
        scratch_types=[
            pltpu.VMEM(((RNG + 1) * FP,), jnp.float32),
            pltpu.VMEM((CHUNK,), jnp.int32),
            pltpu.VMEM((CHUNK,), jnp.int32),
            pltpu.VMEM((CHUNK,), jnp.int32),
            pltpu.VMEM((CHUNK,), jnp.int32),
            pltpu.VMEM((CHUNK + 32,), jnp.int32),
            pltpu.VMEM((CHUNK + 32,), jnp.int32),
            pltpu.VMEM((16, FP), jnp.float32),
            pltpu.VMEM((16, FP), jnp.float32),
            pltpu.SemaphoreType.DMA,
            pltpu.SemaphoreType.DMA,
            pltpu.SemaphoreType.DMA,
            pltpu.SemaphoreType.DMA,
            pltpu.SemaphoreType.DMA,
            pltpu.SemaphoreType.DMA,
        ],
    )
    return kfn(hp, src, dst)


def _post_body(h_ref, p0_ref, p1_ref, p2_ref, p3_ref,
               ws_ref, wne_ref, wno_ref, b_ref, o_ref):
    def unpack(p_ref):
        w = lax.bitcast_convert_type(p_ref[...], jnp.uint32)
        fe = lax.bitcast_convert_type(w << 16, jnp.float32)
        fo = lax.bitcast_convert_type(w & jnp.uint32(0xFFFF0000), jnp.float32)
        return fe, fo

    e0, o0 = unpack(p0_ref)
    e1, o1 = unpack(p1_ref)
    e2, o2 = unpack(p2_ref)
    e3, o3 = unpack(p3_ref)
    ne = jnp.maximum(jnp.maximum(e0, e1), jnp.maximum(e2, e3))
    no = jnp.maximum(jnp.maximum(o0, o1), jnp.maximum(o2, o3))
    o_ref[...] = (h_ref[...] @ ws_ref[...] + ne @ wne_ref[...]
                  + no @ wno_ref[...] + b_ref[...])


def _post(h, partial, W_self, W_neigh, bias):
    blk = 1000
    grid = (N // blk,)
    p = partial.reshape(NUM_EPART, NPAD, FP)
    return pl.pallas_call(
        _post_body,
        grid=grid,
        in_specs=[
            pl.BlockSpec((blk, F), lambda i: (i, 0)),
            pl.BlockSpec((blk, FP), lambda i: (i, 0)),
            pl.BlockSpec((blk, FP), lambda i: (i, 0)),
            pl.BlockSpec((blk, FP), lambda i: (i, 0)),
            pl.BlockSpec((blk, FP), lambda i: (i, 0)),
            pl.BlockSpec((F, O), lambda i: (0, 0)),
            pl.BlockSpec((FP, O), lambda i: (0, 0)),
            pl.BlockSpec((FP, O), lambda i: (0, 0)),
            pl.BlockSpec((1, O), lambda i: (0, 0)),
        ],
        out_specs=pl.BlockSpec((blk, O), lambda i: (i, 0)),
        out_shape=jax.ShapeDtypeStruct((N, O), jnp.float32),
    )(h, p[0, :N], p[1, :N], p[2, :N], p[3, :N],
      W_self, W_neigh[0::2], W_neigh[1::2], bias.reshape(1, O))


@jax.jit
def kernel(x, edge_index, W_pool, b_pool, W_self, W_neigh, bias):
    src = edge_index[0].astype(jnp.int32)
    dst = edge_index[1].astype(jnp.int32)
    h, hp = _pre(x, W_pool, b_pool)
    partial = _segmax(hp, src, dst)
    return _post(h, partial, W_self, W_neigh, bias)


# speedup vs baseline: 2.0199x; 1.0931x over previous
import jax
import jax.numpy as jnp
from jax import lax
from jax.experimental import pallas as pl
from jax.experimental.pallas import tpu as pltpu
from jax.experimental.pallas import tpu_sc as plsc

N = 10000
E = 320000
F = 128
FP = F // 2
O = 64

NUM_RANGES = 8
NUM_EPART = 4
RNG = 1252
NPAD = NUM_RANGES * RNG
EPART = E // NUM_EPART
CHUNK = 4000
NGROUPS = CHUNK // 16
NCHUNK = EPART // CHUNK


def _rne_hi(f):
    u = lax.bitcast_convert_type(f, jnp.uint32)
    r = u + jnp.uint32(0x7FFF) + ((u >> 16) & jnp.uint32(1))
    return r & jnp.uint32(0xFFFF0000)


def _pre_body(x_ref, wpe_ref, wpo_ref, bpe_ref, bpo_ref, h_ref, hpp_ref):
    h = jnp.log(x_ref[...] + 1.0)
    h_ref[...] = h
    hp_e = jnp.maximum(h @ wpe_ref[...] + bpe_ref[...], 0.0)
    hp_o = jnp.maximum(h @ wpo_ref[...] + bpo_ref[...], 0.0)
    word = (_rne_hi(hp_e) >> 16) | _rne_hi(hp_o)
    hpp_ref[...] = lax.bitcast_convert_type(word, jnp.float32)


def _pre(x, W_pool, b_pool):
    blk = 1000
    grid = (N // blk,)
    return pl.pallas_call(
        _pre_body,
        grid=grid,
        in_specs=[
            pl.BlockSpec((blk, F), lambda i: (i, 0)),
            pl.BlockSpec((F, FP), lambda i: (0, 0)),
            pl.BlockSpec((F, FP), lambda i: (0, 0)),
            pl.BlockSpec((1, FP), lambda i: (0, 0)),
            pl.BlockSpec((1, FP), lambda i: (0, 0)),
        ],
        out_specs=[
            pl.BlockSpec((blk, F), lambda i: (i, 0)),
            pl.BlockSpec((blk, FP), lambda i: (i, 0)),
        ],
        out_shape=[
            jax.ShapeDtypeStruct((N, F), jnp.float32),
            jax.ShapeDtypeStruct((N, FP), jnp.float32),
        ],
    )(x, W_pool[:, 0::2], W_pool[:, 1::2],
      b_pool[0::2].reshape(1, FP), b_pool[1::2].reshape(1, FP))


def _segmax_body(hp_hbm, src_hbm, dst_hbm, out_hbm,
                 accum, sbuf0, dbuf0, sbuf1, dbuf1, cpak, rows0, rows1,
                 sem_s0, sem_d0, sem_s1, sem_d1, sem_g0, sem_g1):
    nc = lax.axis_index("c")
    ns = lax.axis_index("s")
    wid = ns * 2 + nc
    rid = wid % NUM_RANGES
    eq = wid // NUM_RANGES
    lo = rid * RNG
    trash = RNG

    zero16 = jnp.zeros((16,), jnp.float32)

    def _z(i, _):
        accum[pl.ds(i * 16, 16)] = zero16
        return 0

    lax.fori_loop(0, (RNG + 1) * FP // 16, _z, 0, unroll=8)

    ebase = eq * EPART

    def stage(c, sb, db, ss, sd):
        off = ebase + c * CHUNK
        pltpu.async_copy(src_hbm.at[pl.ds(off, CHUNK)], sb, ss)
        pltpu.async_copy(dst_hbm.at[pl.ds(off, CHUNK)], db, sd)

    def work(sbuf, dbuf, sem_s, sem_d):
        pltpu.make_async_copy(src_hbm.at[pl.ds(0, CHUNK)], sbuf, sem_s).wait()
        pltpu.make_async_copy(dst_hbm.at[pl.ds(0, CHUNK)], dbuf, sem_d).wait()

        def scan_body(g, n):
            sv = sbuf[pl.ds(g * 16, 16)]
            dv = dbuf[pl.ds(g * 16, 16)]
            d0 = dv - lo
            mi = ((d0 >> 31) + 1) & (((RNG - 1 - d0) >> 31) + 1)
            pos = plsc.cumsum(mi)
            tgt = (CHUNK + 32) + mi * (n + pos - 1 - (CHUNK + 32))
            plsc.store_scatter(cpak, [tgt], sv | (d0 << 14))
            return n + pos[15]

        n = lax.fori_loop(0, NGROUPS, scan_body, jnp.int32(0), unroll=8)

        pad = jnp.full((16,), trash << 14, jnp.int32)
        cpak[pl.ds(n, 16)] = pad
        cpak[pl.ds(n + 16, 16)] = pad
        ngroups = (n + 31) // 32

        def issue(g, rows, sem):
            idxv0 = cpak[pl.ds(g * 32, 16)] & 0x3FFF
            idxv1 = cpak[pl.ds(g * 32 + 16, 16)] & 0x3FFF
            pltpu.async_copy(hp_hbm.at[idxv0], rows.at[pl.ds(0, 16)], sem)
            pltpu.async_copy(hp_hbm.at[idxv1], rows.at[pl.ds(16, 16)], sem)

        def rmw(g, rows, sem):
            pltpu.make_async_copy(hp_hbm.at[cpak[pl.ds(0, 16)] & 0x3FFF],
                                  rows.at[pl.ds(0, 16)], sem).wait()
            pltpu.make_async_copy(hp_hbm.at[cpak[pl.ds(0, 16)] & 0x3FFF],
                                  rows.at[pl.ds(16, 16)], sem).wait()
            for half in range(2):
                dvec = cpak[pl.ds(g * 32 + half * 16, 16)] >> 14
                for j in range(16):
                    d = dvec[j]
                    for f in range(FP // 16):
                        a = plsc.bitcast(accum[pl.ds(d * FP + f * 16, 16)],
                                         jnp.bfloat16)
                        m = plsc.bitcast(
                            rows[half * 16 + j, pl.ds(f * 16, 16)],
                            jnp.bfloat16)
                        accum[pl.ds(d * FP + f * 16, 16)] = plsc.bitcast(
                            jnp.maximum(a, m), jnp.float32)

        @pl.when(ngroups > 0)
        def _():
            issue(0, rows0, sem_g0)

            def pair_body(p, _):
                g0 = p * 2
                g1 = p * 2 + 1

                @pl.when(g1 < ngroups)
                def _():
                    issue(g1, rows1, sem_g1)
                rmw(g0, rows0, sem_g0)

                @pl.when(g1 < ngroups)
                def _():
                    @pl.when(g1 + 1 < ngroups)
                    def _():
                        issue(g1 + 1, rows0, sem_g0)
                    rmw(g1, rows1, sem_g1)
                return 0

            lax.fori_loop(0, (ngroups + 1) // 2, pair_body, 0)

    stage(0, sbuf0, dbuf0, sem_s0, sem_d0)

    def chunk_pair(p, _):
        c0 = p * 2
        stage(c0 + 1, sbuf1, dbuf1, sem_s1, sem_d1)
        work(sbuf0, dbuf0, sem_s0, sem_d0)

        @pl.when(c0 + 2 < NCHUNK)
        def _():
            stage(c0 + 2, sbuf0, dbuf0, sem_s0, sem_d0)
        work(sbuf1, dbuf1, sem_s1, sem_d1)
        return 0

    lax.fori_loop(0, NCHUNK // 2, chunk_pair, 0)

    pltpu.sync_copy(accum.at[pl.ds(0, RNG * FP)],
                    out_hbm.at[eq, pl.ds(lo * FP, RNG * FP)])


def _segmax(hp, src, dst):
    mesh = plsc.VectorSubcoreMesh(core_axis_name="c", subcore_axis_name="s")
    kfn = pl.kernel(
        _segmax_body,
        out_type=jax.ShapeDtypeStruct((NUM_EPART, NPAD * FP), jnp.float32),
        mesh=mesh,
        compiler_params=pltpu.CompilerParams(
            needs_layout_passes=False, use_tc_tiling_on_sc=False),
        scratch_types=[
            pltpu.VMEM(((RNG + 1) * FP,), jnp.float32),
            pltpu.VMEM((CHUNK,), jnp.int32),
            pltpu.VMEM((CHUNK,), jnp.int32),
            pltpu.VMEM((CHUNK,), jnp.int32),
            pltpu.VMEM((CHUNK,), jnp.int32),
            pltpu.VMEM((CHUNK + 64,), jnp.int32),
            pltpu.VMEM((32, FP), jnp.float32),
            pltpu.VMEM((32, FP), jnp.float32),
            pltpu.SemaphoreType.DMA,
            pltpu.SemaphoreType.DMA,
            pltpu.SemaphoreType.DMA,
            pltpu.SemaphoreType.DMA,
            pltpu.SemaphoreType.DMA,
            pltpu.SemaphoreType.DMA,
        ],
    )
    return kfn(hp, src, dst)


def _post_body(h_ref, p0_ref, p1_ref, p2_ref, p3_ref,
               ws_ref, wne_ref, wno_ref, b_ref, o_ref):
    def unpack(p_ref):
        w = lax.bitcast_convert_type(p_ref[...], jnp.uint32)
        fe = lax.bitcast_convert_type(w << 16, jnp.float32)
        fo = lax.bitcast_convert_type(w & jnp.uint32(0xFFFF0000), jnp.float32)
        return fe, fo

    e0, o0 = unpack(p0_ref)
    e1, o1 = unpack(p1_ref)
    e2, o2 = unpack(p2_ref)
    e3, o3 = unpack(p3_ref)
    ne = jnp.maximum(jnp.maximum(e0, e1), jnp.maximum(e2, e3))
    no = jnp.maximum(jnp.maximum(o0, o1), jnp.maximum(o2, o3))
    o_ref[...] = (h_ref[...] @ ws_ref[...] + ne @ wne_ref[...]
                  + no @ wno_ref[...] + b_ref[...])


def _post(h, partial, W_self, W_neigh, bias):
    blk = 1000
    grid = (N // blk,)
    p = partial.reshape(NUM_EPART, NPAD, FP)
    return pl.pallas_call(
        _post_body,
        grid=grid,
        in_specs=[
            pl.BlockSpec((blk, F), lambda i: (i, 0)),
            pl.BlockSpec((blk, FP), lambda i: (i, 0)),
            pl.BlockSpec((blk, FP), lambda i: (i, 0)),
            pl.BlockSpec((blk, FP), lambda i: (i, 0)),
            pl.BlockSpec((blk, FP), lambda i: (i, 0)),
            pl.BlockSpec((F, O), lambda i: (0, 0)),
            pl.BlockSpec((FP, O), lambda i: (0, 0)),
            pl.BlockSpec((FP, O), lambda i: (0, 0)),
            pl.BlockSpec((1, O), lambda i: (0, 0)),
        ],
        out_specs=pl.BlockSpec((blk, O), lambda i: (i, 0)),
        out_shape=jax.ShapeDtypeStruct((N, O), jnp.float32),
    )(h, p[0, :N], p[1, :N], p[2, :N], p[3, :N],
      W_self, W_neigh[0::2], W_neigh[1::2], bias.reshape(1, O))


@jax.jit
def kernel(x, edge_index, W_pool, b_pool, W_self, W_neigh, bias):
    src = edge_index[0].astype(jnp.int32)
    dst = edge_index[1].astype(jnp.int32)
    h, hp = _pre(x, W_pool, b_pool)
    partial = _segmax(hp, src, dst)
    return _post(h, partial, W_self, W_neigh, bias)
